# speedup vs baseline: 1.0216x; 1.0216x over previous
import functools

import jax
import jax.numpy as jnp
from jax.experimental import pallas as pl

N_L = 50000
N_R = 50000
E = 800000


def _head_body(k_ref, x_ref, w1_ref, w2_ref, b_ref, out_ref):
    k = k_ref[...]
    acc = jnp.dot(k, w1_ref[...], preferred_element_type=jnp.float32)
    acc += jnp.dot(k * x_ref[...], w2_ref[...], preferred_element_type=jnp.float32)
    acc += b_ref[...]
    out_ref[...] = jnp.maximum(acc, 0.0)


def _head(K, x, W1, W2, b):
    N, d = K.shape
    h = W1.shape[1]
    BN = 1000
    b2 = b.reshape(1, h)
    return pl.pallas_call(
        _head_body,
        grid=(N // BN,),
        in_specs=[
            pl.BlockSpec((BN, d), lambda i: (i, 0)),
            pl.BlockSpec((BN, d), lambda i: (i, 0)),
            pl.BlockSpec((d, h), lambda i: (0, 0)),
            pl.BlockSpec((d, h), lambda i: (0, 0)),
            pl.BlockSpec((1, h), lambda i: (0, 0)),
        ],
        out_specs=pl.BlockSpec((BN, h), lambda i: (i, 0)),
        out_shape=jax.ShapeDtypeStruct((N, h), jnp.float32),
    )(K, x, W1, W2, b2)


def kernel(l_feat, r_feat, edge_index, edge_weight, W3, b3, W4, b4, W5, b5, W6, b6):
    row = edge_index[0]
    col = edge_index[1]
    w = edge_weight[:, None]

    def spmm(x, src, dst, n_out):
        return jax.ops.segment_sum(w * jnp.take(x, src, axis=0), dst, num_segments=n_out)

    lK1 = spmm(r_feat, col, row, N_L)
    rK1 = spmm(l_feat, row, col, N_R)
    y1 = _head(lK1, l_feat, W3, W4, b3 + b4)
    z1 = _head(rK1, r_feat, W3, W4, b3 + b4)
    lK2 = spmm(z1, col, row, N_L)
    y2 = _head(lK2, y1, W5, W6, b5 + b6)
    return y2
